# baseline (speedup 1.0000x reference)
import jax
import jax.numpy as jnp
from jax import lax
from jax.experimental import pallas as pl
from jax.experimental.pallas import tpu as pltpu
from jax.experimental.pallas import tpu_sc as plsc

_B, _L, _D = 4096, 200, 64
_R = _B * _L
_N = _R * _D
_NC, _NS = 2, 16
_NW = _NC * _NS
_RPW = _R // _NW
_C = 256
_CW = _C * _D
_G = _RPW // _C
_LANES = 16


def _sc_body(left_hbm, lm_hbm, right_hbm, rm_hbm, out_hbm,
             lb0, rb0, ob0, lm0, rm0,
             lb1, rb1, ob1, lm1, rm1,
             in0, in1, ou0, ou1):
    wid = lax.axis_index("s") * _NC + lax.axis_index("c")
    base = wid * _RPW

    slots = ((lb0, rb0, ob0, lm0, rm0, in0, ou0),
             (lb1, rb1, ob1, lm1, rm1, in1, ou1))

    def issue_in(g, slot):
        lb, rb, _, lm, rm, isem, _ = slots[slot]
        row0 = base + g * _C
        w0 = row0 * _D
        pltpu.make_async_copy(left_hbm.at[pl.ds(w0, _CW)], lb, isem).start()
        pltpu.make_async_copy(right_hbm.at[pl.ds(w0, _CW)], rb, isem).start()
        pltpu.make_async_copy(lm_hbm.at[pl.ds(row0, _C)], lm, isem).start()
        pltpu.make_async_copy(rm_hbm.at[pl.ds(row0, _C)], rm, isem).start()

    def wait_in(slot):
        lb, rb, _, lm, rm, isem, _ = slots[slot]
        pltpu.make_async_copy(left_hbm.at[pl.ds(0, _CW)], lb, isem).wait()
        pltpu.make_async_copy(right_hbm.at[pl.ds(0, _CW)], rb, isem).wait()
        pltpu.make_async_copy(lm_hbm.at[pl.ds(0, _C)], lm, isem).wait()
        pltpu.make_async_copy(rm_hbm.at[pl.ds(0, _C)], rm, isem).wait()

    def issue_out(g, slot):
        _, _, ob, _, _, _, osem = slots[slot]
        w0 = (base + g * _C) * _D
        pltpu.make_async_copy(ob, out_hbm.at[pl.ds(w0, _CW)], osem).start()

    def wait_out(slot):
        _, _, ob, _, _, _, osem = slots[slot]
        pltpu.make_async_copy(ob, out_hbm.at[pl.ds(0, _CW)], osem).wait()

    def compute(slot):
        lb, rb, ob, lm, rm, _, _ = slots[slot]
        dnums = lax.GatherDimensionNumbers(
            offset_dims=(), collapsed_slice_dims=(0,), start_index_map=(0,))

        def group_body(grp, carry):
            r0 = grp * _LANES
            lmg = lm[pl.ds(r0, _LANES)]
            rmg = rm[pl.ds(r0, _LANES)]
            for j in range(_LANES):
                idxv = jnp.full((_LANES, 1), j, dtype=jnp.int32)
                lmv = lax.gather(lmg, idxv, dnums, slice_sizes=(1,),
                                 mode=lax.GatherScatterMode.PROMISE_IN_BOUNDS)
                rmv = lax.gather(rmg, idxv, dnums, slice_sizes=(1,),
                                 mode=lax.GatherScatterMode.PROMISE_IN_BOUNDS)
                w0 = (r0 + j) * _D
                for k in range(_D // _LANES):
                    lv = lb[pl.ds(w0 + k * _LANES, _LANES)]
                    rv = rb[pl.ds(w0 + k * _LANES, _LANES)]
                    ob[pl.ds(w0 + k * _LANES, _LANES)] = (lv - rv * rmv) * lmv
            return carry

        lax.fori_loop(0, _C // _LANES, group_body, 0)

    issue_in(0, 0)
    issue_in(1, 1)

    def pair_body(gp, carry):
        for slot in (0, 1):
            g = 2 * gp + slot

            @pl.when(gp > 0)
            def _():
                wait_out(slot)

            wait_in(slot)
            compute(slot)
            issue_out(g, slot)

            @pl.when(gp + 1 < _G // 2)
            def _():
                issue_in(g + 2, slot)

        return carry

    lax.fori_loop(0, _G // 2, pair_body, 0)
    wait_out(0)
    wait_out(1)


_sc_call = pl.kernel(
    _sc_body,
    out_type=jax.ShapeDtypeStruct((_N,), jnp.float32),
    mesh=plsc.VectorSubcoreMesh(core_axis_name="c", subcore_axis_name="s"),
    scratch_types=(
        [pltpu.VMEM((_CW,), jnp.float32)] * 3
        + [pltpu.VMEM((_C,), jnp.float32)] * 2
    ) * 2
    + [pltpu.SemaphoreType.DMA] * 4,
)


def kernel(left, left_mask, right, right_mask):
    lf = left.reshape(_N)
    rf = right.reshape(_N)
    lmf = left_mask.reshape(_R).astype(jnp.float32)
    rmf = right_mask.reshape(_R).astype(jnp.float32)
    out = _sc_call(lf, lmf, rf, rmf)
    return out.reshape(_B, _L, _D)
